# baseline (speedup 1.0000x reference)
import functools

import jax
import jax.numpy as jnp
from jax import lax
from jax.experimental import pallas as pl
from jax.experimental.pallas import tpu as pltpu
from jax.experimental.pallas import tpu_sc as plsc

C0 = 0.28209479177387814
C1 = 0.4886025119029199
C2A = 1.0925484305920792
C2B = 0.31539156525252005
C2C = 0.5462742152960396
CUTOFF = 5.0
N_M = 9
F = 288
WNODE = 320
NWIN = 160
WPW = 5
CH = 1024
LAST_FULL = 155
PART_ROWS = 80

_COS_COEF = (1.0, -4.934802200544679, 4.0587121264167685,
             -1.3352627688545895, 0.23533063035889320,
             -0.025806891390014061, 1.9295743094039231e-03,
             -1.0463810492484570e-04, 4.3030695870329470e-06)


def _sc_body(vx_hbm, vy_hbm, vz_hbm, idx_hbm, wb_hbm, zeros_hbm, out_hbm,
             table, bvx, bvy, bvz, bidx, bwb, sem0, sem1):
    wid = lax.axis_index("s") * 2 + lax.axis_index("c")
    pltpu.sync_copy(wb_hbm, bwb)
    iota = lax.broadcasted_iota(jnp.int32, (16,), 0)

    def getb(k):
        return bwb[pl.ds(k, 16)][0]

    sigma = CUTOFF / 8.0
    inv2s2 = 1.0 / (2.0 * sigma * sigma)

    def do_window(w, row0, partial):
        b_lo = getb(w)
        b_hi = getb(w + 1)
        node_base = w * WNODE

        @pl.when(b_hi > b_lo)
        def _():
            t_lo = b_lo // CH
            t_hi = (b_hi + CH - 1) // CH

            def dmas(t, slot, sem):
                e0 = t * CH
                o = slot * CH
                return [
                    pltpu.make_async_copy(vx_hbm.at[pl.ds(e0, CH)],
                                          bvx.at[pl.ds(o, CH)], sem),
                    pltpu.make_async_copy(vy_hbm.at[pl.ds(e0, CH)],
                                          bvy.at[pl.ds(o, CH)], sem),
                    pltpu.make_async_copy(vz_hbm.at[pl.ds(e0, CH)],
                                          bvz.at[pl.ds(o, CH)], sem),
                    pltpu.make_async_copy(idx_hbm.at[pl.ds(e0, CH)],
                                          bidx.at[pl.ds(o, CH)], sem),
                ]

            for d in dmas(t_lo, 0, sem0):
                d.start()
            pltpu.sync_copy(zeros_hbm, table)

            def chunk_body(i, carry):
                t = t_lo + i
                slot = lax.rem(i, 2)
                sl = slot * CH
                sem = sem0

                @pl.when(slot == 0)
                def _():
                    for d in dmas(t, 0, sem0):
                        d.wait()

                @pl.when(slot == 1)
                def _():
                    for d in dmas(t, 1, sem1):
                        d.wait()

                @pl.when(t + 1 < t_hi)
                def _():
                    @pl.when(slot == 0)
                    def _():
                        for d in dmas(t + 1, 1, sem1):
                            d.start()

                    @pl.when(slot == 1)
                    def _():
                        for d in dmas(t + 1, 0, sem0):
                            d.start()

                e0 = t * CH

                def group_body(j, carry2):
                    s = sl + j * 16
                    x = bvx[pl.ds(s, 16)]
                    y = bvy[pl.ds(s, 16)]
                    z = bvz[pl.ds(s, 16)]
                    idr = bidx[pl.ds(s, 16)]
                    eg = e0 + (s - sl) + iota
                    valid = (eg >= b_lo) & (eg < b_hi)
                    r2 = jnp.maximum(x * x + y * y + z * z, 1e-24)
                    ih = 0x5F3759DF - lax.shift_right_logical(
                        lax.bitcast_convert_type(r2, jnp.int32), 1)
                    q = lax.bitcast_convert_type(ih, jnp.float32)
                    h = 0.5 * r2
                    q = q * (1.5 - h * q * q)
                    q = q * (1.5 - h * q * q)
                    q = q * (1.5 - h * q * q)
                    r = r2 * q
                    xs, ys, zs = x * q, y * q, z * q
                    t1 = jnp.minimum(r, CUTOFF) * (1.0 / CUTOFF)
                    u = t1 * t1
                    c = jnp.full_like(u, _COS_COEF[8])
                    for cc in _COS_COEF[7::-1]:
                        c = c * u + cc
                    fc = 0.5 * c + 0.5
                    ds = [r - (k * (CUTOFF / 7.0)) for k in range(8)]
                    gs = [jnp.exp((d * d) * (-inv2s2)) for d in ds]
                    shs = [
                        jnp.full_like(r, C0) * fc,
                        (C1 * fc) * ys, (C1 * fc) * zs, (C1 * fc) * xs,
                        (C2A * fc) * (xs * ys), (C2A * fc) * (ys * zs),
                        fc * (C2B * (3.0 * zs * zs - 1.0)),
                        (C2A * fc) * (xs * zs),
                        fc * (C2C * (xs * xs - ys * ys)),
                    ]
                    nl = lax.shift_right_logical(idr, 2) - node_base
                    off0 = nl * F + jnp.bitwise_and(idr, 3) * 8
                    for m in range(N_M):
                        offm = off0 + (m * 32)
                        for k in range(8):
                            plsc.addupdate_scatter(
                                table, [offm + k], shs[m] * gs[k],
                                mask=valid)
                    return carry2

                lax.fori_loop(0, CH // 16, group_body, 0)
                return carry

            lax.fori_loop(0, t_hi - t_lo, chunk_body, 0)

        @pl.when(jnp.logical_not(partial))
        def _():
            pltpu.sync_copy(table, out_hbm.at[pl.ds(row0 * F, WNODE * F)])

        @pl.when(partial)
        def _():
            pltpu.sync_copy(table.at[pl.ds(0, PART_ROWS * F)],
                            out_hbm.at[pl.ds(row0 * F, PART_ROWS * F)])

    for j in range(WPW):
        w = wid * WPW + j
        row0 = jnp.minimum(w * WNODE, 50000 - PART_ROWS)

        @pl.when(w <= LAST_FULL + 1)
        def _():
            do_window(w, w * WNODE * 0 + row0, w == LAST_FULL + 1)


def _mix_body(raw_ref, mix_ref, out_ref):
    out_ref[...] = lax.dot_general(
        raw_ref[...].astype(jnp.bfloat16), mix_ref[...],
        (((1,), (0,)), ((), ())), preferred_element_type=jnp.float32)


@jax.jit
def kernel(vectors, radial_mix, density_indices):
    e = vectors.shape[0]
    n_nodes = 50000
    e_pad = ((e + CH - 1) // CH) * CH
    idx = density_indices.astype(jnp.int32)
    idx_p = jnp.pad(idx, (0, e_pad - e), constant_values=jnp.int32(0x3FFFFFF8))
    v_p = jnp.pad(vectors, ((0, e_pad - e), (0, 0)))
    vT = v_p.T
    vx, vy, vz = vT[0], vT[1], vT[2]
    bounds = jnp.arange(NWIN + 1, dtype=jnp.int32) * (WNODE * 4)
    wb = jnp.searchsorted(idx_p, bounds).astype(jnp.int32)
    wb = jnp.pad(wb, (0, 192 - (NWIN + 1)))
    zeros = jnp.zeros((WNODE * F,), jnp.float32)

    mesh = plsc.VectorSubcoreMesh(core_axis_name="c", subcore_axis_name="s",
                                  num_cores=2, num_subcores=16)
    raw = pl.kernel(
        _sc_body,
        out_type=jax.ShapeDtypeStruct((n_nodes * F,), jnp.float32),
        mesh=mesh,
        compiler_params=pltpu.CompilerParams(use_tc_tiling_on_sc=False,
                                             needs_layout_passes=False),
        scratch_types=[
            pltpu.VMEM((WNODE * F,), jnp.float32),
            pltpu.VMEM((2 * CH,), jnp.float32),
            pltpu.VMEM((2 * CH,), jnp.float32),
            pltpu.VMEM((2 * CH,), jnp.float32),
            pltpu.VMEM((2 * CH,), jnp.int32),
            pltpu.VMEM((192,), jnp.int32),
            pltpu.SemaphoreType.DMA,
            pltpu.SemaphoreType.DMA,
        ],
    )(vx, vy, vz, idx_p, wb, zeros)
    raw = raw.reshape(n_nodes, F)

    bigmix = (jnp.kron(jnp.eye(N_M * 4, dtype=jnp.float32), radial_mix)
              .astype(jnp.bfloat16))
    rb = 256
    nblk = (n_nodes + rb - 1) // rb
    out2 = pl.pallas_call(
        _mix_body,
        grid=(nblk,),
        in_specs=[pl.BlockSpec((rb, F), lambda i: (i, 0)),
                  pl.BlockSpec((F, F), lambda i: (0, 0))],
        out_specs=pl.BlockSpec((rb, F), lambda i: (i, 0)),
        out_shape=jax.ShapeDtypeStruct((n_nodes, F), jnp.float32),
    )(raw, bigmix)
    return out2.reshape(n_nodes, N_M, 32)


# speedup vs baseline: 1.0021x; 1.0021x over previous
import functools

import jax
import jax.numpy as jnp
from jax import lax
from jax.experimental import pallas as pl
from jax.experimental.pallas import tpu as pltpu
from jax.experimental.pallas import tpu_sc as plsc

C0 = 0.28209479177387814
C1 = 0.4886025119029199
C2A = 1.0925484305920792
C2B = 0.31539156525252005
C2C = 0.5462742152960396
CUTOFF = 5.0
N_M = 9
F = 288
N_NODES = 50000
N1 = 30016
WN = 64
B = 512
NBLK_TC = N1 // WN
WNODE = 320
NWIN = 63
WPW = 2
CH = 1024
LAST_FULL = 61
PART_ROWS = 144

_COS_COEF = (1.0, -4.934802200544679, 4.0587121264167685,
             -1.3352627688545895, 0.23533063035889320,
             -0.025806891390014061, 1.9295743094039231e-03,
             -1.0463810492484570e-04, 4.3030695870329470e-06)


def _sc_body(vx_hbm, vy_hbm, vz_hbm, idx_hbm, wb_hbm, zeros_hbm, out_hbm,
             table, bvx, bvy, bvz, bidx, bwb):
    wid = lax.axis_index("s") * 2 + lax.axis_index("c")
    pltpu.sync_copy(wb_hbm, bwb)
    iota = lax.broadcasted_iota(jnp.int32, (16,), 0)

    def getb(k):
        return bwb[pl.ds(k, 16)][0]

    sigma = CUTOFF / 8.0
    inv2s2 = 1.0 / (2.0 * sigma * sigma)

    def do_window(w, row0, partial):
        b_lo = getb(w)
        b_hi = getb(w + 1)
        node_base = N1 + w * WNODE

        @pl.when(b_hi > b_lo)
        def _():
            pltpu.sync_copy(zeros_hbm, table)
            t_lo = b_lo // CH
            t_hi = (b_hi + CH - 1) // CH

            def chunk_body(t, carry):
                e0 = t * CH
                pltpu.sync_copy(vx_hbm.at[pl.ds(e0, CH)], bvx)
                pltpu.sync_copy(vy_hbm.at[pl.ds(e0, CH)], bvy)
                pltpu.sync_copy(vz_hbm.at[pl.ds(e0, CH)], bvz)
                pltpu.sync_copy(idx_hbm.at[pl.ds(e0, CH)], bidx)

                def group_body(j, carry2):
                    s = j * 16
                    x = bvx[pl.ds(s, 16)]
                    y = bvy[pl.ds(s, 16)]
                    z = bvz[pl.ds(s, 16)]
                    idr = bidx[pl.ds(s, 16)]
                    eg = e0 + s + iota
                    valid = (eg >= b_lo) & (eg < b_hi)
                    r2 = jnp.maximum(x * x + y * y + z * z, 1e-24)
                    ih = 0x5F3759DF - lax.shift_right_logical(
                        lax.bitcast_convert_type(r2, jnp.int32), 1)
                    q = lax.bitcast_convert_type(ih, jnp.float32)
                    h = 0.5 * r2
                    q = q * (1.5 - h * q * q)
                    q = q * (1.5 - h * q * q)
                    q = q * (1.5 - h * q * q)
                    r = r2 * q
                    xs, ys, zs = x * q, y * q, z * q
                    t1 = jnp.minimum(r, CUTOFF) * (1.0 / CUTOFF)
                    u = t1 * t1
                    c = jnp.full_like(u, _COS_COEF[8])
                    for cc in _COS_COEF[7::-1]:
                        c = c * u + cc
                    fc = 0.5 * c + 0.5
                    dsr = [r - (k * (CUTOFF / 7.0)) for k in range(8)]
                    gs = [jnp.exp((d * d) * (-inv2s2)) for d in dsr]
                    shs = [
                        jnp.full_like(r, C0) * fc,
                        (C1 * fc) * ys, (C1 * fc) * zs, (C1 * fc) * xs,
                        (C2A * fc) * (xs * ys), (C2A * fc) * (ys * zs),
                        fc * (C2B * (3.0 * zs * zs - 1.0)),
                        (C2A * fc) * (xs * zs),
                        fc * (C2C * (xs * xs - ys * ys)),
                    ]
                    nl = lax.shift_right_logical(idr, 2) - node_base
                    off0 = nl * F + jnp.bitwise_and(idr, 3) * 8
                    for m in range(N_M):
                        offm = off0 + (m * 32)
                        for k in range(8):
                            plsc.addupdate_scatter(
                                table, [offm + k], shs[m] * gs[k],
                                mask=valid)
                    return carry2

                lax.fori_loop(0, CH // 16, group_body, 0)
                return carry

            lax.fori_loop(t_lo, t_hi, chunk_body, 0)

        @pl.when(jnp.logical_not(partial))
        def _():
            pltpu.sync_copy(table, out_hbm.at[pl.ds(row0 * F, WNODE * F)])

        @pl.when(partial)
        def _():
            pltpu.sync_copy(table.at[pl.ds(0, PART_ROWS * F)],
                            out_hbm.at[pl.ds(row0 * F, PART_ROWS * F)])

    nsc = N_NODES - N1
    for j in range(WPW):
        w = wid * WPW + j
        row0 = jnp.minimum(w * WNODE, nsc - PART_ROWS)

        @pl.when(w <= LAST_FULL + 1)
        def _():
            do_window(w, row0, w == LAST_FULL + 1)


def _tc_body(e_bounds_ref, vx_ref, vy_ref, vz_ref, idx_ref, bigmix_ref,
             out_ref, acc_ref):
    i = pl.program_id(0)
    e_lo = e_bounds_ref[i]
    e_hi = e_bounds_ref[i + 1]
    c_lo = e_lo // B
    c_hi = (e_hi + B - 1) // B
    nchunk = vx_ref.shape[0]
    node_base = i * WN
    sigma = CUTOFF / 8.0
    inv2s2 = 1.0 / (2.0 * sigma * sigma)
    bf = jnp.bfloat16

    def contrib(c, valid):
        vx = vx_ref[c]
        vy = vy_ref[c]
        vz = vz_ref[c]
        idr = idx_ref[c]
        r2 = jnp.maximum(vx * vx + vy * vy + vz * vz, 1e-24)
        rinv = jax.lax.rsqrt(r2)
        r = r2 * rinv
        xs, ys, zs = vx * rinv, vy * rinv, vz * rinv
        fc = 0.5 * (jnp.cos(jnp.pi * jnp.clip(r, 0.0, CUTOFF) / CUTOFF) + 1.0)
        r8 = jnp.broadcast_to(r, (8, B))
        mus = (jax.lax.broadcasted_iota(jnp.int32, (8, B), 0)
               .astype(jnp.float32) * (CUTOFF / 7.0))
        g = jnp.exp(-((r8 - mus) ** 2) * inv2s2) \
            * jnp.broadcast_to(fc, (8, B))
        g16 = jnp.concatenate([g, g], axis=0)
        sp16 = jnp.broadcast_to(jnp.bitwise_and(idr, 3), (16, B))
        row_par = jax.lax.shift_right_logical(
            jax.lax.broadcasted_iota(jnp.int32, (16, B), 0), 3)
        zero16 = jnp.zeros((16, B), jnp.float32)
        gs_a = jnp.where(sp16 == row_par, g16, zero16).astype(bf)
        gs_b = jnp.where(sp16 == row_par + 2, g16, zero16).astype(bf)
        shs = [
            jnp.full_like(r, C0),
            C1 * ys, C1 * zs, C1 * xs,
            C2A * xs * ys, C2A * ys * zs, C2B * (3.0 * zs * zs - 1.0),
            C2A * xs * zs, C2C * (xs * xs - ys * ys),
        ]
        pieces = []
        for m in range(N_M):
            sh16 = jnp.broadcast_to(shs[m].astype(bf), (16, B))
            pieces.append(sh16 * gs_a)
            pieces.append(sh16 * gs_b)
        feats = jnp.concatenate(pieces, axis=0)
        nl = jax.lax.shift_right_logical(idr, 2) - node_base
        nl = jnp.where(valid, nl, -1)
        iota = jax.lax.broadcasted_iota(jnp.int32, (WN, B), 0)
        oh = (iota == nl).astype(bf)
        return jax.lax.dot_general(
            oh, feats, (((1,), (1,)), ((), ())),
            preferred_element_type=jnp.float32)

    acc_ref[...] = jnp.zeros((WN, F), jnp.float32)
    n_t = (c_hi - c_lo + 1) // 2

    def body(t, _):
        c0 = c_lo + 2 * t
        c1 = jnp.minimum(c0 + 1, nchunk - 1)
        d0 = contrib(c0, True)
        d1 = contrib(c1, c0 + 1 < c_hi)
        acc_ref[...] = acc_ref[...] + d0 + d1
        return 0

    jax.lax.fori_loop(0, n_t, body, 0)
    out_ref[...] = jax.lax.dot_general(
        acc_ref[...].astype(bf), bigmix_ref[...], (((1,), (0,)), ((), ())),
        preferred_element_type=jnp.float32)


def _mix_body(raw_ref, mix_ref, out_ref):
    out_ref[...] = lax.dot_general(
        raw_ref[...].astype(jnp.bfloat16), mix_ref[...],
        (((1,), (0,)), ((), ())), preferred_element_type=jnp.float32)


@jax.jit
def kernel(vectors, radial_mix, density_indices):
    e = vectors.shape[0]
    e_pad = ((e + 1023) // 1024) * 1024
    idx = density_indices.astype(jnp.int32)
    idx_p = jnp.pad(idx, (0, e_pad - e), constant_values=jnp.int32(0x3FFFFFF8))
    v_p = jnp.pad(vectors, ((0, e_pad - e), (0, 0)))
    vT = v_p.T
    vx, vy, vz = vT[0], vT[1], vT[2]
    nchunk = e_pad // B
    vx3 = vx.reshape(nchunk, 1, B)
    vy3 = vy.reshape(nchunk, 1, B)
    vz3 = vz.reshape(nchunk, 1, B)
    idx3 = idx_p.reshape(nchunk, 1, B)
    bigmix = (jnp.kron(jnp.eye(N_M * 4, dtype=jnp.float32), radial_mix)
              .astype(jnp.bfloat16))

    wb_sc = jnp.searchsorted(
        idx_p, (N1 + jnp.arange(NWIN + 1, dtype=jnp.int32) * WNODE) * 4
    ).astype(jnp.int32)
    wb_sc = jnp.pad(wb_sc, (0, 88 - (NWIN + 1)))
    zeros = jnp.zeros((WNODE * F,), jnp.float32)
    nsc = N_NODES - N1
    mesh = plsc.VectorSubcoreMesh(core_axis_name="c", subcore_axis_name="s",
                                  num_cores=2, num_subcores=16)
    raw = pl.kernel(
        _sc_body,
        out_type=jax.ShapeDtypeStruct((nsc * F,), jnp.float32),
        mesh=mesh,
        compiler_params=pltpu.CompilerParams(use_tc_tiling_on_sc=False,
                                             needs_layout_passes=False),
        scratch_types=[
            pltpu.VMEM((WNODE * F,), jnp.float32),
            pltpu.VMEM((CH,), jnp.float32),
            pltpu.VMEM((CH,), jnp.float32),
            pltpu.VMEM((CH,), jnp.float32),
            pltpu.VMEM((CH,), jnp.int32),
            pltpu.VMEM((88,), jnp.int32),
        ],
    )(vx, vy, vz, idx_p, wb_sc, zeros)
    raw = raw.reshape(nsc, F)

    eb_tc = jnp.searchsorted(
        idx_p, jnp.arange(NBLK_TC + 1, dtype=jnp.int32) * (WN * 4)
    ).astype(jnp.int32)
    edge_spec = pl.BlockSpec((nchunk, 1, B), lambda i, s: (0, 0, 0))
    out_tc = pl.pallas_call(
        _tc_body,
        grid_spec=pltpu.PrefetchScalarGridSpec(
            num_scalar_prefetch=1,
            grid=(NBLK_TC,),
            in_specs=[edge_spec, edge_spec, edge_spec, edge_spec,
                      pl.BlockSpec((F, F), lambda i, s: (0, 0))],
            out_specs=pl.BlockSpec((WN, F), lambda i, s: (i, 0)),
            scratch_shapes=[pltpu.VMEM((WN, F), jnp.float32)],
        ),
        out_shape=jax.ShapeDtypeStruct((N1, F), jnp.float32),
    )(eb_tc, vx3, vy3, vz3, idx3, bigmix)

    rb = 256
    nblk_m = (nsc + rb - 1) // rb
    out_sc = pl.pallas_call(
        _mix_body,
        grid=(nblk_m,),
        in_specs=[pl.BlockSpec((rb, F), lambda i: (i, 0)),
                  pl.BlockSpec((F, F), lambda i: (0, 0))],
        out_specs=pl.BlockSpec((rb, F), lambda i: (i, 0)),
        out_shape=jax.ShapeDtypeStruct((nsc, F), jnp.float32),
    )(raw, bigmix)

    out = jnp.concatenate([out_tc, out_sc], axis=0)
    return out.reshape(N_NODES, N_M, 32)
